# baseline (speedup 1.0000x reference)
import functools

import jax
import jax.numpy as jnp
from jax import lax
from jax.experimental import pallas as pl
from jax.experimental.pallas import tpu as pltpu
from jax.experimental.pallas import tpu_sc as plsc

N = 100000
D = 256
S = 1024
NV = D // 16

NC, NS = 2, 16
NW = NC * NS
CHUNK = 32
CHUNKS_PER_W = 100
ROWS_PER_W = CHUNK * CHUNKS_PER_W
N_PAD = NW * ROWS_PER_W
FLUSH = 128
STG = 1280

BLK = 512
NBLK = N_PAD // BLK


def _phi_body(x_ref, w1_ref, b1_ref, w2_ref, b2_ref, out_ref):
    h = jnp.maximum(
        jnp.dot(x_ref[...], w1_ref[...],
                preferred_element_type=jnp.float32) + b1_ref[...], 0.0)
    out_ref[...] = jnp.dot(h, w2_ref[...],
                           preferred_element_type=jnp.float32) + b2_ref[...]


def _phi(x_pad, W1, b1, W2, b2):
    wspec = pl.BlockSpec((D, D), lambda i: (0, 0))
    bspec = pl.BlockSpec((D,), lambda i: (0,))
    return pl.pallas_call(
        _phi_body,
        grid=(NBLK,),
        in_specs=[pl.BlockSpec((BLK, D), lambda i: (i, 0)),
                  wspec, bspec, wspec, bspec],
        out_specs=pl.BlockSpec((BLK, D), lambda i: (i, 0)),
        out_shape=jax.ShapeDtypeStruct((N_PAD, D), jnp.float32),
    )(x_pad, W1, b1, W2, b2)


def _segsum_body(xph_hbm, idx_hbm, zeros_hbm, stg_hbm, rows_v, idx_v, flush_v,
                 acc_v):
    cid = lax.axis_index("c")
    sid = lax.axis_index("s")
    wid = sid * NC + cid
    r0 = wid * ROWS_PER_W
    stg0 = wid * STG

    pltpu.sync_copy(idx_hbm.at[pl.ds(r0, ROWS_PER_W)],
                    idx_v.at[pl.ds(0, ROWS_PER_W)])
    pltpu.sync_copy(zeros_hbm, flush_v)
    for z in range(STG // FLUSH):
        pltpu.sync_copy(zeros_hbm, stg_hbm.at[pl.ds(stg0 + z * FLUSH, FLUSH)])
    seg0 = idx_v[pl.ds(0, 16)][0]
    base8 = (seg0 // 8) * 8

    z16 = jnp.zeros((16,), jnp.float32)
    for j in range(NV):
        acc_v[pl.ds(j * 16, 16)] = z16

    def spill_dma(wb_old):
        pltpu.sync_copy(
            flush_v,
            stg_hbm.at[pl.ds(pl.multiple_of(stg0 + wb_old, 8), FLUSH)])
        pltpu.sync_copy(zeros_hbm, flush_v)

    def chunk(c, carry):
        pltpu.sync_copy(xph_hbm.at[pl.ds(r0 + c * CHUNK, CHUNK)], rows_v)
        cur, wb = carry
        for g in range(CHUNK // 16):
            segs = idx_v[pl.ds(pl.multiple_of(c * CHUNK + g * 16, 16), 16)]
            uniform = segs[0] == segs[15]

            lane_info = []
            for lane in range(16):
                s = segs[lane]
                ch = s != cur
                pos_d = cur - base8
                sp = jnp.logical_and(ch, pos_d - wb >= FLUSH)
                wb_new = jnp.where(sp, (pos_d // FLUSH) * FLUSH, wb)
                lane_info.append((ch, sp, pos_d, wb, wb_new))
                wb = wb_new
                cur = s

            @pl.when(uniform)
            def _fast(g=g, info=lane_info[0]):
                ch, sp, pos_d, wb_old, wb_new = info

                @pl.when(sp)
                def _spill():
                    spill_dma(wb_old)

                @pl.when(ch)
                def _dump():
                    prel = pos_d - wb_new
                    for j in range(NV):
                        flush_v[prel, pl.ds(j * 16, 16)] = \
                            acc_v[pl.ds(j * 16, 16)]
                        acc_v[pl.ds(j * 16, 16)] = z16

                for j in range(NV):
                    gs = rows_v[g * 16, pl.ds(j * 16, 16)]
                    for lane in range(1, 16):
                        gs = gs + rows_v[g * 16 + lane, pl.ds(j * 16, 16)]
                    acc_v[pl.ds(j * 16, 16)] += gs

            @pl.when(jnp.logical_not(uniform))
            def _slow(g=g, info=tuple(lane_info)):
                for lane in range(16):
                    ch, sp, pos_d, wb_old, wb_new = info[lane]

                    @pl.when(sp)
                    def _spill(wb_old=wb_old):
                        spill_dma(wb_old)

                    @pl.when(ch)
                    def _dump(pos_d=pos_d, wb_new=wb_new):
                        prel = pos_d - wb_new
                        for j in range(NV):
                            flush_v[prel, pl.ds(j * 16, 16)] = \
                                acc_v[pl.ds(j * 16, 16)]
                            acc_v[pl.ds(j * 16, 16)] = z16

                    for j in range(NV):
                        acc_v[pl.ds(j * 16, 16)] += \
                            rows_v[g * 16 + lane, pl.ds(j * 16, 16)]

        return (cur, wb)

    cur, wbase = lax.fori_loop(0, CHUNKS_PER_W, chunk, (seg0, jnp.int32(0)))

    pos_d = cur - base8
    sp = pos_d - wbase >= FLUSH

    @pl.when(sp)
    def _final_spill():
        spill_dma(wbase)

    wbase = jnp.where(sp, (pos_d // FLUSH) * FLUSH, wbase)
    prel = pos_d - wbase
    for j in range(NV):
        flush_v[prel, pl.ds(j * 16, 16)] = acc_v[pl.ds(j * 16, 16)]
    pltpu.sync_copy(flush_v,
                    stg_hbm.at[pl.ds(pl.multiple_of(stg0 + wbase, 8), FLUSH)])


def _segsum(x_phi, idx_pad):
    zeros = jnp.zeros((FLUSH, D), jnp.float32)
    k = pl.kernel(
        _segsum_body,
        out_type=jax.ShapeDtypeStruct((NW * STG, D), jnp.float32),
        mesh=plsc.VectorSubcoreMesh(core_axis_name="c", subcore_axis_name="s"),
        scratch_types=[
            pltpu.VMEM((CHUNK, D), jnp.float32),
            pltpu.VMEM((ROWS_PER_W + 16,), jnp.int32),
            pltpu.VMEM((FLUSH, D), jnp.float32),
            pltpu.VMEM((D,), jnp.float32),
        ],
    )
    return k(x_phi, idx_pad, zeros)


ACC_ROWS = 2304


def _combine_rho_body(base8_ref, stg_ref, wr1_ref, br1_ref, wr2_ref, br2_ref,
                      out_ref, acc_ref):
    w = pl.program_id(0)

    @pl.when(w == 0)
    def _init():
        acc_ref[...] = jnp.zeros_like(acc_ref)

    off = pl.multiple_of(base8_ref[w], 8)
    acc_ref[pl.ds(off, STG), :] += stg_ref[0]

    @pl.when(w == NW - 1)
    def _rho():
        x_sum = acc_ref[pl.ds(0, S), :]
        h2 = jnp.maximum(
            jnp.dot(x_sum, wr1_ref[...],
                    preferred_element_type=jnp.float32) + br1_ref[...], 0.0)
        out_ref[...] = jnp.dot(h2, wr2_ref[...],
                               preferred_element_type=jnp.float32) + br2_ref[...]


def _combine_rho(staged, base8, Wr1, br1, Wr2, br2):
    return pl.pallas_call(
        _combine_rho_body,
        grid=(NW,),
        in_specs=[
            pl.BlockSpec(memory_space=pltpu.SMEM),
            pl.BlockSpec((1, STG, D), lambda w: (w, 0, 0)),
            pl.BlockSpec((D, D), lambda w: (0, 0)),
            pl.BlockSpec((D,), lambda w: (0,)),
            pl.BlockSpec((D, D), lambda w: (0, 0)),
            pl.BlockSpec((D,), lambda w: (0,)),
        ],
        out_specs=pl.BlockSpec((S, D), lambda w: (0, 0)),
        out_shape=jax.ShapeDtypeStruct((S, D), jnp.float32),
        scratch_shapes=[pltpu.VMEM((ACC_ROWS, D), jnp.float32)],
    )(base8, staged.reshape(NW, STG, D), Wr1, br1, Wr2, br2)


@jax.jit
def _run(x, idx_i32, W_phi1, b_phi1, W_phi2, b_phi2,
         W_rho1, b_rho1, W_rho2, b_rho2):
    x_pad = jnp.pad(x, ((0, N_PAD - N), (0, 0)))
    idx_pad = jnp.pad(idx_i32, (0, N_PAD - N), constant_values=S)
    base8 = (idx_pad[:: ROWS_PER_W] // 8) * 8
    x_phi = _phi(x_pad, W_phi1, b_phi1, W_phi2, b_phi2)
    staged = _segsum(x_phi, idx_pad)
    return _combine_rho(staged, base8, W_rho1, b_rho1, W_rho2, b_rho2)


def kernel(x, batch_idx, W_phi1, b_phi1, W_phi2, b_phi2,
           W_rho1, b_rho1, W_rho2, b_rho2):
    idx_i32 = batch_idx.astype(jnp.int32)
    return _run(x, idx_i32, W_phi1, b_phi1, W_phi2, b_phi2,
                W_rho1, b_rho1, W_rho2, b_rho2)


# speedup vs baseline: 8.4125x; 8.4125x over previous
import functools

import jax
import jax.numpy as jnp
from jax.experimental import pallas as pl
from jax.experimental.pallas import tpu as pltpu

N = 100000
D = 256
S = 1024
BLK = 512
N_PAD = ((N + BLK - 1) // BLK) * BLK
NBLK = N_PAD // BLK
W = 64
ACC_ROWS = S + W + 8


def _fused_body(base8_ref, last_ref, idx_ref, x_ref,
                w1_ref, b1_ref, w2_ref, b2_ref,
                wr1_ref, br1_ref, wr2_ref, br2_ref, out_ref, acc_ref):
    i = pl.program_id(0)

    @pl.when(i == 0)
    def _init():
        acc_ref[...] = jnp.zeros_like(acc_ref)

    h = jnp.maximum(
        jnp.dot(x_ref[...], w1_ref[...],
                preferred_element_type=jnp.float32) + b1_ref[...], 0.0)
    xp = jnp.dot(h, w2_ref[...],
                 preferred_element_type=jnp.float32) + b2_ref[...]

    idx = idx_ref[0, 0, :]
    base8 = base8_ref[i]
    overflow = last_ref[i] - base8 >= W

    seg_iota = base8 + jax.lax.broadcasted_iota(jnp.int32, (W, BLK), 0)
    onehot = (seg_iota == idx[None, :]).astype(jnp.float32)
    part = jnp.dot(onehot, xp, preferred_element_type=jnp.float32)
    off = pl.multiple_of(base8, 8)
    acc_ref[pl.ds(off, W), :] += part

    @pl.when(overflow)
    def _full():
        iota_s = jax.lax.broadcasted_iota(jnp.int32, (S, BLK), 0)
        beyond = idx >= base8 + W
        ohf = jnp.logical_and(iota_s == idx[None, :],
                              beyond[None, :]).astype(jnp.float32)
        acc_ref[pl.ds(0, S), :] += jnp.dot(
            ohf, xp, preferred_element_type=jnp.float32)

    @pl.when(i == NBLK - 1)
    def _rho():
        h2 = jnp.maximum(
            jnp.dot(acc_ref[pl.ds(0, S), :], wr1_ref[...],
                    preferred_element_type=jnp.float32) + br1_ref[...], 0.0)
        out_ref[...] = jnp.dot(h2, wr2_ref[...],
                               preferred_element_type=jnp.float32) + br2_ref[...]


@jax.jit
def _run(x, idx_i32, W_phi1, b_phi1, W_phi2, b_phi2,
         W_rho1, b_rho1, W_rho2, b_rho2):
    x_pad = jnp.pad(x, ((0, N_PAD - N), (0, 0)))
    idx_pad = jnp.pad(idx_i32, (0, N_PAD - N), constant_values=S)
    idx3 = idx_pad.reshape(NBLK, 1, BLK)
    base8 = (idx_pad[:: BLK] // 8) * 8
    last = idx_pad[BLK - 1:: BLK]

    wspec = pl.BlockSpec((D, D), lambda i: (0, 0))
    bspec = pl.BlockSpec((D,), lambda i: (0,))
    out = pl.pallas_call(
        _fused_body,
        grid=(NBLK,),
        in_specs=[
            pl.BlockSpec(memory_space=pltpu.SMEM),
            pl.BlockSpec(memory_space=pltpu.SMEM),
            pl.BlockSpec((1, 1, BLK), lambda i: (i, 0, 0)),
            pl.BlockSpec((BLK, D), lambda i: (i, 0)),
            wspec, bspec, wspec, bspec,
            wspec, bspec,
            pl.BlockSpec((D, D), lambda i: (0, 0)),
            pl.BlockSpec((D,), lambda i: (0,)),
        ],
        out_specs=pl.BlockSpec((S, D), lambda i: (0, 0)),
        out_shape=jax.ShapeDtypeStruct((S, D), jnp.float32),
        scratch_shapes=[pltpu.VMEM((ACC_ROWS, D), jnp.float32)],
    )(base8, last, idx3, x_pad, W_phi1, b_phi1, W_phi2, b_phi2,
      W_rho1, b_rho1, W_rho2, b_rho2)
    return out


def kernel(x, batch_idx, W_phi1, b_phi1, W_phi2, b_phi2,
           W_rho1, b_rho1, W_rho2, b_rho2):
    idx_i32 = batch_idx.astype(jnp.int32)
    return _run(x, idx_i32, W_phi1, b_phi1, W_phi2, b_phi2,
                W_rho1, b_rho1, W_rho2, b_rho2)
